# baseline (speedup 1.0000x reference)
import functools
import math

import jax
import jax.numpy as jnp
from jax import lax
from jax.experimental import pallas as pl
from jax.experimental.pallas import tpu as pltpu
from jax.experimental.pallas import tpu_sc as plsc

NC = 2
NS = 16
NW = NC * NS
LANES = 16
CH = 128
NBUF = 6


def _emb_kernel(B, V, D):
    per_w = B // NW
    nch = per_w // CH
    scale = math.sqrt(D)
    mesh = plsc.VectorSubcoreMesh(
        core_axis_name="c", subcore_axis_name="s", num_cores=NC, num_subcores=NS
    )
    assert (nch - 8) % NBUF == 0 and nch > NBUF + 4

    @functools.partial(
        pl.kernel,
        mesh=mesh,
        out_type=jax.ShapeDtypeStruct((B // 2, 2 * D), jnp.float32),
        compiler_params=pltpu.CompilerParams(use_tc_tiling_on_sc=True),
        scratch_types=[
            pltpu.VMEM((NBUF, CH), jnp.int32),
            pltpu.VMEM((NBUF, CH), jnp.int32),
            pltpu.VMEM((NBUF, CH, 2 * D), jnp.float32),
        ]
        + [pltpu.SemaphoreType.DMA] * (3 * NBUF),
    )
    def k(tok_hbm, table_hbm, out_hbm, tok_v, pidx_v, rows_v, *sems):
        isem = sems[:NBUF]
        gsem = sems[NBUF : 2 * NBUF]
        wsem = sems[2 * NBUF :]
        wid = lax.axis_index("s") * NC + lax.axis_index("c")
        tbase = wid * per_w
        obase = wid * (per_w // 2)

        def fire_idx(j, b):
            pltpu.async_copy(tok_hbm.at[pl.ds(tbase + j * CH, CH)], tok_v.at[b], isem[b])

        def wait_idx(b):
            pltpu.make_async_copy(tok_hbm.at[pl.ds(0, CH)], tok_v.at[b], isem[b]).wait()

        def fire_gather(b):
            for t in range(CH // LANES):
                sl = pl.ds(t * LANES, LANES)
                pidx_v[b, sl] = lax.shift_right_logical(tok_v[b, sl], 1)
            pltpu.async_copy(table_hbm.at[pidx_v.at[b]], rows_v.at[b], gsem[b])

        def wait_gather(b):
            pltpu.make_async_copy(
                out_hbm.at[pl.ds(0, CH)], rows_v.at[b], gsem[b]
            ).wait()

        def fire_write(j, b):
            pltpu.async_copy(
                rows_v.at[b, pl.ds(0, CH // 2)],
                out_hbm.at[pl.ds(obase + j * (CH // 2), CH // 2)],
                wsem[b],
            )

        def wait_write(b):
            pltpu.make_async_copy(
                out_hbm.at[pl.ds(0, CH // 2)],
                rows_v.at[b, pl.ds(0, CH // 2)],
                wsem[b],
            ).wait()

        def compact_scale(b):
            def g_body(g, c):
                tv = tok_v[b, pl.ds(g * LANES, LANES)]
                for l in range(LANES):
                    src_row = g * LANES + l
                    dst_row = g * (LANES // 2) + l // 2
                    off = (tv[l] & 1) * D
                    for t in range(D // LANES):
                        dst = pl.ds((l & 1) * D + t * LANES, LANES)
                        src = pl.ds(off + t * LANES, LANES)
                        rows_v[b, dst_row, dst] = rows_v[b, src_row, src] * scale
                return c

            lax.fori_loop(0, CH // LANES, g_body, 0)

        def process(j, b):
            wait_gather(b)
            compact_scale(b)
            fire_write(j, b)

        for j in range(4):
            fire_idx(j, j)
        for j in range(2):
            wait_idx(j)
            fire_gather(j)

        for j in range(4):
            process(j, j)
            fire_idx(j + 4, (j + 4) % NBUF)
            wait_idx((j + 2) % NBUF)
            fire_gather((j + 2) % NBUF)

        def body(m, carry):
            j0 = 4 + m * NBUF
            for u in range(NBUF):
                j = j0 + u
                process(j, (4 + u) % NBUF)
                fire_idx(j + 4, (4 + u + 4) % NBUF)
                b2 = (4 + u + 2) % NBUF
                wait_idx(b2)
                wait_write(b2)
                fire_gather(b2)
            return carry

        lax.fori_loop(0, (nch - 8) // NBUF, body, 0)

        for u in range(4):
            j = nch - 4 + u
            process(j, j % NBUF)
            if u < 2:
                wait_idx((j + 2) % NBUF)
                wait_write((j + 2) % NBUF)
                fire_gather((j + 2) % NBUF)

        for b in range(NBUF):
            wait_write(b)

    return k


def kernel(tokens, table):
    B0, T = tokens.shape
    V, D = table.shape
    B = B0 * T
    assert B % (NW * CH) == 0 and D % LANES == 0 and V % 2 == 0 and CH % (2 * LANES) == 0
    tok = tokens.reshape(B).astype(jnp.int32)
    tab2 = table.reshape(V // 2, 2 * D)
    out = _emb_kernel(B, V, D)(tok, tab2)
    return out.reshape(B0, T, D)
